# Initial kernel scaffold; baseline (speedup 1.0000x reference)
import jax
import jax.numpy as jnp
from jax.experimental import pallas as pl


def kernel(x, edge_index, W1, b1, W2, b2, W3, b3, bn_gamma, bn_beta, W_out, b_out):
    raise NotImplementedError("write your pallas kernel here")


# speedup vs baseline: 6.5660x; 6.5660x over previous
import functools

import jax
import jax.numpy as jnp
from jax import lax
from jax.experimental import pallas as pl
from jax.experimental.pallas import tpu as pltpu
from jax.experimental.pallas import tpu_sc as plsc

N = 10000
E = 160000
D = 256
H = 128
NC = 2
NS = 16
CH = 128

EP = ((E + NC * NS * CH - 1) // (NC * NS * CH)) * (NC * NS * CH)
ECHUNKS = EP // CH
SPMM_CPT = ECHUNKS // NS
DEG_CPT = ECHUNKS // (NC * NS)
NPAD = 10240
ROWS_PT = NPAD // NS

MM_BLOCK = 1000
GRID = N // MM_BLOCK


def _sc_mesh():
  return plsc.VectorSubcoreMesh(core_axis_name="c", subcore_axis_name="s",
                                num_cores=NC, num_subcores=NS)


def _deg_body(dst2_hbm, ones_hbm, zeros_hbm, out_hbm,
              acc_sp, dst_v, ones_v, sem):
  c = lax.axis_index("c")
  s = lax.axis_index("s")
  rbase = pl.multiple_of(s * ROWS_PT, 8)
  pltpu.sync_copy(zeros_hbm.at[pl.ds(rbase, ROWS_PT)],
                  acc_sp.at[pl.ds(rbase, ROWS_PT)])
  pltpu.sync_copy(ones_hbm, ones_v)
  base = pl.multiple_of((c * NS + s) * DEG_CPT, 8)
  pltpu.async_copy(dst2_hbm.at[pl.ds(base, DEG_CPT)], dst_v, sem).wait()
  plsc.subcore_barrier()

  def step(j, _):
    pltpu.sync_copy(ones_v, acc_sp.at[dst_v.at[j]], add=True)
    return ()

  lax.fori_loop(0, DEG_CPT, step, (), unroll=False)
  plsc.subcore_barrier()
  pltpu.sync_copy(acc_sp.at[pl.ds(rbase, ROWS_PT)],
                  out_hbm.at[c].at[pl.ds(rbase, ROWS_PT)])


def _deg_call(dst2, ones, zeros):
  return pl.kernel(
      _deg_body,
      out_type=jax.ShapeDtypeStruct((NC, NPAD, 16), jnp.float32),
      mesh=_sc_mesh(),
      scratch_types=[
          pltpu.VMEM_SHARED((NPAD, 16), jnp.float32),
          pltpu.VMEM((DEG_CPT, CH), jnp.int32),
          pltpu.VMEM((CH, 16), jnp.float32),
          pltpu.SemaphoreType.DMA,
      ],
  )(dst2, ones, zeros)


def _spmm_body(hs_hbm, src2_hbm, dst2_hbm, out_hbm,
               acc_sp, src_v, dst_v, rows_v, sem, semg):
  c = lax.axis_index("c")
  s = lax.axis_index("s")
  rbase = pl.multiple_of(s * ROWS_PT, 8)
  cbase = pl.multiple_of(s * SPMM_CPT, 8)
  pltpu.async_copy(src2_hbm.at[pl.ds(cbase, SPMM_CPT)], src_v, sem).wait()
  pltpu.async_copy(dst2_hbm.at[pl.ds(cbase, SPMM_CPT)], dst_v, sem).wait()
  pltpu.sync_copy(hs_hbm.at[c].at[pl.ds(rbase, ROWS_PT)],
                  acc_sp.at[pl.ds(rbase, ROWS_PT)])
  plsc.subcore_barrier()

  def step(j, _):
    pltpu.async_copy(hs_hbm.at[c].at[src_v.at[j]], rows_v, semg).wait()
    pltpu.sync_copy(rows_v, acc_sp.at[dst_v.at[j]], add=True)
    return ()

  lax.fori_loop(0, SPMM_CPT, step, (), unroll=False)
  plsc.subcore_barrier()
  pltpu.sync_copy(acc_sp.at[pl.ds(rbase, ROWS_PT)],
                  out_hbm.at[c].at[pl.ds(rbase, ROWS_PT)])


def _spmm_call(hs, src2, dst2):
  return pl.kernel(
      _spmm_body,
      out_type=jax.ShapeDtypeStruct((NC, NPAD, H), jnp.float32),
      mesh=_sc_mesh(),
      scratch_types=[
          pltpu.VMEM_SHARED((NPAD, H), jnp.float32),
          pltpu.VMEM((SPMM_CPT, CH), jnp.int32),
          pltpu.VMEM((SPMM_CPT, CH), jnp.int32),
          pltpu.VMEM((CH, H), jnp.float32),
          pltpu.SemaphoreType.DMA,
          pltpu.SemaphoreType.DMA,
      ],
  )(hs, src2, dst2)


def _dinv_from(degp_blk):
  deg = degp_blk[0, :, 0:1] + degp_blk[1, :, 0:1] + 1.0
  return lax.rsqrt(deg)


def _layer1_body(x_ref, deg_ref, w_ref, hs_ref, sx_ref, sx2_ref):
  i = pl.program_id(0)
  x = x_ref[...]
  dinv = _dinv_from(deg_ref[...])
  u = jnp.dot(x, w_ref[...], preferred_element_type=jnp.float32)
  hs = dinv * u
  hs_ref[0] = hs[:, :H]
  hs_ref[1] = hs[:, H:]

  @pl.when(i == 0)
  def _():
    sx_ref[...] = jnp.zeros_like(sx_ref)
    sx2_ref[...] = jnp.zeros_like(sx2_ref)

  sx_ref[...] += jnp.sum(x, axis=0, keepdims=True)
  sx2_ref[...] += jnp.sum(x * x, axis=0, keepdims=True)


def _layer1_call(x, deg2, w):
  return pl.pallas_call(
      _layer1_body,
      grid=(GRID,),
      in_specs=[
          pl.BlockSpec((MM_BLOCK, D), lambda i: (i, 0)),
          pl.BlockSpec((NC, MM_BLOCK, 16), lambda i: (0, i, 0)),
          pl.BlockSpec((D, D), lambda i: (0, 0)),
      ],
      out_specs=[
          pl.BlockSpec((NC, MM_BLOCK, H), lambda i: (0, i, 0)),
          pl.BlockSpec((1, D), lambda i: (0, 0)),
          pl.BlockSpec((1, D), lambda i: (0, 0)),
      ],
      out_shape=[
          jax.ShapeDtypeStruct((NC, NPAD, H), jnp.float32),
          jax.ShapeDtypeStruct((1, D), jnp.float32),
          jax.ShapeDtypeStruct((1, D), jnp.float32),
      ],
  )(x, deg2, w)


def _layer_mid_body(agg_ref, deg_ref, b_ref, w_ref,
                    h_ref, hs_ref, s_ref, s2_ref):
  i = pl.program_id(0)
  dinv = _dinv_from(deg_ref[...])
  agg = jnp.concatenate([agg_ref[0], agg_ref[1]], axis=1)
  h = jnp.maximum(dinv * agg + b_ref[...], 0.0)
  h_ref[...] = h
  u = jnp.dot(h, w_ref[...], preferred_element_type=jnp.float32)
  hs = dinv * u
  hs_ref[0] = hs[:, :H]
  hs_ref[1] = hs[:, H:]

  @pl.when(i == 0)
  def _():
    s_ref[...] = jnp.zeros_like(s_ref)
    s2_ref[...] = jnp.zeros_like(s2_ref)

  s_ref[...] += jnp.sum(h, axis=0, keepdims=True)
  s2_ref[...] += jnp.sum(h * h, axis=0, keepdims=True)


def _layer_mid_call(agg, deg2, b, w):
  return pl.pallas_call(
      _layer_mid_body,
      grid=(GRID,),
      in_specs=[
          pl.BlockSpec((NC, MM_BLOCK, H), lambda i: (0, i, 0)),
          pl.BlockSpec((NC, MM_BLOCK, 16), lambda i: (0, i, 0)),
          pl.BlockSpec((1, D), lambda i: (0, 0)),
          pl.BlockSpec((D, D), lambda i: (0, 0)),
      ],
      out_specs=[
          pl.BlockSpec((MM_BLOCK, D), lambda i: (i, 0)),
          pl.BlockSpec((NC, MM_BLOCK, H), lambda i: (0, i, 0)),
          pl.BlockSpec((1, D), lambda i: (0, 0)),
          pl.BlockSpec((1, D), lambda i: (0, 0)),
      ],
      out_shape=[
          jax.ShapeDtypeStruct((N, D), jnp.float32),
          jax.ShapeDtypeStruct((NC, NPAD, H), jnp.float32),
          jax.ShapeDtypeStruct((1, D), jnp.float32),
          jax.ShapeDtypeStruct((1, D), jnp.float32),
      ],
  )(agg, deg2, b, w)


def _layer_last_body(agg_ref, deg_ref, b_ref, h_ref, s_ref, s2_ref):
  i = pl.program_id(0)
  dinv = _dinv_from(deg_ref[...])
  agg = jnp.concatenate([agg_ref[0], agg_ref[1]], axis=1)
  h = jnp.maximum(dinv * agg + b_ref[...], 0.0)
  h_ref[...] = h

  @pl.when(i == 0)
  def _():
    s_ref[...] = jnp.zeros_like(s_ref)
    s2_ref[...] = jnp.zeros_like(s2_ref)

  s_ref[...] += jnp.sum(h, axis=0, keepdims=True)
  s2_ref[...] += jnp.sum(h * h, axis=0, keepdims=True)


def _layer_last_call(agg, deg2, b):
  return pl.pallas_call(
      _layer_last_body,
      grid=(GRID,),
      in_specs=[
          pl.BlockSpec((NC, MM_BLOCK, H), lambda i: (0, i, 0)),
          pl.BlockSpec((NC, MM_BLOCK, 16), lambda i: (0, i, 0)),
          pl.BlockSpec((1, D), lambda i: (0, 0)),
      ],
      out_specs=[
          pl.BlockSpec((MM_BLOCK, D), lambda i: (i, 0)),
          pl.BlockSpec((1, D), lambda i: (0, 0)),
          pl.BlockSpec((1, D), lambda i: (0, 0)),
      ],
      out_shape=[
          jax.ShapeDtypeStruct((N, D), jnp.float32),
          jax.ShapeDtypeStruct((1, D), jnp.float32),
          jax.ShapeDtypeStruct((1, D), jnp.float32),
      ],
  )(agg, deg2, b)


def _out_body(x_ref, h1_ref, h2_ref, h3_ref, sums_ref, sumsq_ref,
              gamma_ref, beta_ref, wout_ref, bout_ref, o_ref):
  mean = sums_ref[...] * (1.0 / N)
  var = sumsq_ref[...] * (1.0 / N) - mean * mean
  scale = gamma_ref[...] * lax.rsqrt(var + 1e-5)
  shift = beta_ref[...] - mean * scale
  hc = jnp.concatenate(
      [x_ref[...], h1_ref[...], h2_ref[...], h3_ref[...]], axis=1)
  hn = hc * scale
  w = wout_ref[...]
  out = jnp.dot(hn, w, preferred_element_type=jnp.float32)
  corr = jnp.dot(shift, w, preferred_element_type=jnp.float32)
  o_ref[...] = out + corr + bout_ref[...]


def _out_call(x, h1, h2, h3, sums, sumsq, gamma, beta, w_out, b_out):
  blk = pl.BlockSpec((MM_BLOCK, D), lambda i: (i, 0))
  row4 = pl.BlockSpec((1, 4 * D), lambda i: (0, 0))
  return pl.pallas_call(
      _out_body,
      grid=(GRID,),
      in_specs=[
          blk, blk, blk, blk,
          row4, row4, row4, row4,
          pl.BlockSpec((4 * D, D), lambda i: (0, 0)),
          pl.BlockSpec((1, D), lambda i: (0, 0)),
      ],
      out_specs=blk,
      out_shape=jax.ShapeDtypeStruct((N, D), jnp.float32),
  )(x, h1, h2, h3, sums, sumsq, gamma, beta, w_out, b_out)


def kernel(x, edge_index, W1, b1, W2, b2, W3, b3, bn_gamma, bn_beta,
           W_out, b_out):
  src = edge_index[0]
  dst = edge_index[1]
  pad = EP - E
  src2 = jnp.concatenate([src, jnp.zeros((pad,), jnp.int32)]).reshape(ECHUNKS, CH)
  dst2 = jnp.concatenate([dst, jnp.full((pad,), N, jnp.int32)]).reshape(ECHUNKS, CH)

  ones16 = jnp.zeros((CH, 16), jnp.float32).at[:, 0].set(1.0)
  zeros16 = jnp.zeros((NPAD, 16), jnp.float32)
  deg2 = _deg_call(dst2, ones16, zeros16)

  b1r = b1.reshape(1, D)
  b2r = b2.reshape(1, D)
  b3r = b3.reshape(1, D)

  hs1, sx, sx2 = _layer1_call(x, deg2, W1)
  agg1 = _spmm_call(hs1, src2, dst2)
  h1, hs2, s1, s12 = _layer_mid_call(agg1, deg2, b1r, W2)
  agg2 = _spmm_call(hs2, src2, dst2)
  h2, hs3, s2, s22 = _layer_mid_call(agg2, deg2, b2r, W3)
  agg3 = _spmm_call(hs3, src2, dst2)
  h3, s3, s32 = _layer_last_call(agg3, deg2, b3r)

  sums = jnp.concatenate([sx, s1, s2, s3], axis=1)
  sumsq = jnp.concatenate([sx2, s12, s22, s32], axis=1)
  gam = bn_gamma.reshape(1, 4 * D)
  bet = bn_beta.reshape(1, 4 * D)
  return _out_call(x, h1, h2, h3, sums, sumsq, gam, bet, W_out,
                   b_out.reshape(1, D))
